# Initial kernel scaffold; baseline (speedup 1.0000x reference)
import jax
import jax.numpy as jnp
from jax.experimental import pallas as pl


def kernel(x_path, x_omic1, x_omic2, x_omic3, x_omic4, x_omic5, x_omic6, edge_index, edge_latent, params):
    raise NotImplementedError("write your pallas kernel here")


# speedup vs baseline: 16.1720x; 16.1720x over previous
import functools

import jax
import jax.numpy as jnp
from jax import lax
from jax.experimental import pallas as pl

N = 4096
D = 256
T = 4102
T_PAD = 4224
ROW_BLK = 512
ATT_BLK = 128

_f32 = jnp.float32


def _relu6(x):
    return jnp.clip(x, 0.0, 6.0)


def _elu(x):
    return jnp.where(x > 0, x, jnp.exp(jnp.minimum(x, 0.0)) - 1.0)


def _build_adj(src, dst):
    flat = dst.astype(jnp.int32) * N + src.astype(jnp.int32)
    a = jnp.zeros((N * N,), _f32).at[flat].add(1.0)
    return a.reshape(N, N)


def _dinv_kernel(a_ref, o_ref):
    o_ref[...] = lax.rsqrt(jnp.sum(a_ref[...], axis=1, keepdims=True))


def _compute_dinv(a):
    grid = N // ROW_BLK
    return pl.pallas_call(
        _dinv_kernel,
        grid=(grid,),
        in_specs=[pl.BlockSpec((ROW_BLK, N), lambda i: (i, 0))],
        out_specs=pl.BlockSpec((ROW_BLK, 1), lambda i: (i, 0)),
        out_shape=jax.ShapeDtypeStruct((N, 1), _f32),
    )(a)


def _pathfc_kernel(x_ref, w1_ref, b1_ref, w2_ref, b2_ref, o_ref):
    h = jnp.dot(x_ref[...], w1_ref[...], preferred_element_type=_f32)
    h = _relu6(h + b1_ref[...])
    h = jnp.dot(h, w2_ref[...], preferred_element_type=_f32)
    o_ref[...] = _relu6(h + b2_ref[...])


def _path_fc(x, w1, b1, w2, b2):
    grid = N // ROW_BLK
    return pl.pallas_call(
        _pathfc_kernel,
        grid=(grid,),
        in_specs=[
            pl.BlockSpec((ROW_BLK, 768), lambda i: (i, 0)),
            pl.BlockSpec((768, D), lambda i: (0, 0)),
            pl.BlockSpec((1, D), lambda i: (0, 0)),
            pl.BlockSpec((D, D), lambda i: (0, 0)),
            pl.BlockSpec((1, D), lambda i: (0, 0)),
        ],
        out_specs=pl.BlockSpec((ROW_BLK, D), lambda i: (i, 0)),
        out_shape=jax.ShapeDtypeStruct((N, D), _f32),
    )(x, w1, b1, w2, b2)


def _gen_kernel(x_ref, w1_ref, b1_ref, w2_ref, b2_ref, o_ref):
    h = jnp.dot(x_ref[0], w1_ref[0], preferred_element_type=_f32)
    h = _elu(h + b1_ref[0])
    h = jnp.dot(h, w2_ref[0], preferred_element_type=_f32)
    o_ref[0] = _elu(h + b2_ref[0])


def _genomics(xs, w1s, b1s, w2s, b2s):
    return pl.pallas_call(
        _gen_kernel,
        grid=(6,),
        in_specs=[
            pl.BlockSpec((1, 1, 640), lambda i: (i, 0, 0)),
            pl.BlockSpec((1, 640, 1024), lambda i: (i, 0, 0)),
            pl.BlockSpec((1, 1, 1024), lambda i: (i, 0, 0)),
            pl.BlockSpec((1, 1024, D), lambda i: (i, 0, 0)),
            pl.BlockSpec((1, 1, D), lambda i: (i, 0, 0)),
        ],
        out_specs=pl.BlockSpec((1, 1, D), lambda i: (i, 0, 0)),
        out_shape=jax.ShapeDtypeStruct((6, 1, D), _f32),
    )(xs, w1s, b1s, w2s, b2s)


def _scale_mm_kernel(h_ref, w_ref, dv_ref, o_ref):
    z = jnp.dot(h_ref[...], w_ref[...], preferred_element_type=_f32)
    o_ref[...] = z * dv_ref[...]


def _gcn_z(h, w, dinv):
    grid = N // ROW_BLK
    din, dout = w.shape
    return pl.pallas_call(
        _scale_mm_kernel,
        grid=(grid,),
        in_specs=[
            pl.BlockSpec((ROW_BLK, din), lambda i: (i, 0)),
            pl.BlockSpec((din, dout), lambda i: (0, 0)),
            pl.BlockSpec((ROW_BLK, 1), lambda i: (i, 0)),
        ],
        out_specs=pl.BlockSpec((ROW_BLK, dout), lambda i: (i, 0)),
        out_shape=jax.ShapeDtypeStruct((N, dout), _f32),
    )(h, w, dinv)


def _gcn_agg_kernel(a_ref, z_ref, dv_ref, b_ref, o_ref, *, relu):
    acc = jnp.dot(a_ref[...], z_ref[...], preferred_element_type=_f32)
    out = acc * dv_ref[...] + b_ref[...]
    if relu:
        out = jnp.maximum(out, 0.0)
    o_ref[...] = out


def _gcn_agg(a, z, dinv, b, relu):
    grid = N // ROW_BLK
    dout = z.shape[1]
    return pl.pallas_call(
        functools.partial(_gcn_agg_kernel, relu=relu),
        grid=(grid,),
        in_specs=[
            pl.BlockSpec((ROW_BLK, N), lambda i: (i, 0)),
            pl.BlockSpec((N, dout), lambda i: (0, 0)),
            pl.BlockSpec((ROW_BLK, 1), lambda i: (i, 0)),
            pl.BlockSpec((1, dout), lambda i: (0, 0)),
        ],
        out_specs=pl.BlockSpec((ROW_BLK, dout), lambda i: (i, 0)),
        out_shape=jax.ShapeDtypeStruct((N, dout), _f32),
    )(a, z, dinv, b)


def _gcn_layer(a, h, w, b, dinv, relu):
    z = _gcn_z(h, w, dinv)
    return _gcn_agg(a, z, dinv, b, relu)


def _qkv_kernel(t_ref, w_ref, b_ref, o_ref):
    o_ref[...] = (
        jnp.dot(t_ref[...], w_ref[...], preferred_element_type=_f32) + b_ref[...]
    )


def _qkv_proj(tok, wqkv, bqkv):
    grid = T_PAD // ATT_BLK
    return pl.pallas_call(
        _qkv_kernel,
        grid=(grid,),
        in_specs=[
            pl.BlockSpec((ATT_BLK, D), lambda i: (i, 0)),
            pl.BlockSpec((D, 3 * D), lambda i: (0, 0)),
            pl.BlockSpec((1, 3 * D), lambda i: (0, 0)),
        ],
        out_specs=pl.BlockSpec((ATT_BLK, 3 * D), lambda i: (i, 0)),
        out_shape=jax.ShapeDtypeStruct((T_PAD, 3 * D), _f32),
    )(tok, wqkv, bqkv)


def _att_kernel(tok_ref, qkv_blk_ref, qkv_ref, wo_ref, bo_ref,
                fw1_ref, fb1_ref, fw2_ref, fb2_ref, g_ref, be_ref, o_ref):
    heads = 4
    dh = D // heads
    scale = 1.0 / jnp.sqrt(jnp.float32(dh))
    col = lax.broadcasted_iota(jnp.int32, (ATT_BLK, T_PAD), 1)
    outs = []
    for h in range(heads):
        q = qkv_blk_ref[:, h * dh:(h + 1) * dh]
        k = qkv_ref[:, D + h * dh:D + (h + 1) * dh]
        v = qkv_ref[:, 2 * D + h * dh:2 * D + (h + 1) * dh]
        s = lax.dot_general(q, k, (((1,), (1,)), ((), ())),
                            preferred_element_type=_f32) * scale
        s = jnp.where(col < T, s, -1e30)
        m = jnp.max(s, axis=1, keepdims=True)
        p = jnp.exp(s - m)
        l = jnp.sum(p, axis=1, keepdims=True)
        o = jnp.dot(p, v, preferred_element_type=_f32) / l
        outs.append(o)
    o = jnp.concatenate(outs, axis=1)
    x = tok_ref[...] + jnp.dot(o, wo_ref[...], preferred_element_type=_f32) + bo_ref[...]
    hh = jnp.maximum(
        jnp.dot(x, fw1_ref[...], preferred_element_type=_f32) + fb1_ref[...], 0.0)
    x = x + jnp.dot(hh, fw2_ref[...], preferred_element_type=_f32) + fb2_ref[...]
    mu = jnp.mean(x, axis=1, keepdims=True)
    xc = x - mu
    var = jnp.mean(xc * xc, axis=1, keepdims=True)
    o_ref[...] = xc * lax.rsqrt(var + 1e-5) * g_ref[...] + be_ref[...]


def _attention(tok, qkv, wo, bo, fw1, fb1, fw2, fb2, g, b):
    grid = T_PAD // ATT_BLK
    return pl.pallas_call(
        _att_kernel,
        grid=(grid,),
        in_specs=[
            pl.BlockSpec((ATT_BLK, D), lambda i: (i, 0)),
            pl.BlockSpec((ATT_BLK, 3 * D), lambda i: (i, 0)),
            pl.BlockSpec((T_PAD, 3 * D), lambda i: (0, 0)),
            pl.BlockSpec((D, D), lambda i: (0, 0)),
            pl.BlockSpec((1, D), lambda i: (0, 0)),
            pl.BlockSpec((D, 1024), lambda i: (0, 0)),
            pl.BlockSpec((1, 1024), lambda i: (0, 0)),
            pl.BlockSpec((1024, D), lambda i: (0, 0)),
            pl.BlockSpec((1, D), lambda i: (0, 0)),
            pl.BlockSpec((1, D), lambda i: (0, 0)),
            pl.BlockSpec((1, D), lambda i: (0, 0)),
        ],
        out_specs=pl.BlockSpec((ATT_BLK, D), lambda i: (i, 0)),
        out_shape=jax.ShapeDtypeStruct((T_PAD, D), _f32),
    )(tok, qkv, qkv, wo, bo, fw1, fb1, fw2, fb2, g, b)


def _head_kernel(x_ref, mmw_ref, mmb_ref, cw_ref, cb_ref, o_ref):
    x = x_ref[...]
    row = lax.broadcasted_iota(jnp.int32, (T_PAD, D), 0)
    paths = jnp.sum(jnp.where(row < 6, x, 0.0), axis=0, keepdims=True) / 6.0
    wsi = jnp.sum(jnp.where((row >= 6) & (row < T), x, 0.0), axis=0,
                  keepdims=True) / jnp.float32(T - 6)
    cat = jnp.concatenate([paths, wsi], axis=1)
    fusion = jnp.maximum(
        jnp.dot(cat, mmw_ref[...], preferred_element_type=_f32) + mmb_ref[...], 0.0)
    logits = jnp.dot(fusion, cw_ref[...], preferred_element_type=_f32) + cb_ref[...]
    o_ref[...] = jnp.broadcast_to(logits, (8, 128))


def _head(x, mmw, mmb, cw, cb):
    return pl.pallas_call(
        _head_kernel,
        grid=(1,),
        in_specs=[
            pl.BlockSpec((T_PAD, D), lambda i: (0, 0)),
            pl.BlockSpec((2 * D, 128), lambda i: (0, 0)),
            pl.BlockSpec((1, 128), lambda i: (0, 0)),
            pl.BlockSpec((128, 128), lambda i: (0, 0)),
            pl.BlockSpec((1, 128), lambda i: (0, 0)),
        ],
        out_specs=pl.BlockSpec((8, 128), lambda i: (0, 0)),
        out_shape=jax.ShapeDtypeStruct((8, 128), _f32),
    )(x, mmw, mmb, cw, cb)


def kernel(x_path, x_omic1, x_omic2, x_omic3, x_omic4, x_omic5, x_omic6,
           edge_index, edge_latent, params):
    p = params
    omics = [x_omic1, x_omic2, x_omic3, x_omic4, x_omic5, x_omic6]

    loop = jnp.arange(N, dtype=edge_index.dtype)
    src = jnp.concatenate([edge_index[0], edge_latent[0], loop])
    dst = jnp.concatenate([edge_index[1], edge_latent[1], loop])
    a = _build_adj(src, dst)
    dinv = _compute_dinv(a)

    x = x_path.reshape(N, 768)
    path_feat = _path_fc(x, p["pW1"], p["pb1"].reshape(1, D),
                         p["pW2"], p["pb2"].reshape(1, D))

    xs = jnp.stack([jnp.pad(xo, (0, 640 - xo.shape[0])) for xo in omics])[:, None, :]
    w1s = jnp.stack([jnp.pad(p["gW1_%d" % i], ((0, 640 - p["gW1_%d" % i].shape[0]), (0, 0)))
                     for i in range(6)])
    b1s = jnp.stack([p["gb1_%d" % i] for i in range(6)])[:, None, :]
    w2s = jnp.stack([p["gW2_%d" % i] for i in range(6)])
    b2s = jnp.stack([p["gb2_%d" % i] for i in range(6)])[:, None, :]
    gen = _genomics(xs, w1s, b1s, w2s, b2s).reshape(6, D)

    h = _gcn_layer(a, path_feat, p["cW1"], p["cb1"].reshape(1, -1), dinv, relu=True)
    h = _gcn_layer(a, h, p["cW2"], p["cb2"].reshape(1, -1), dinv, relu=True)
    h = _gcn_layer(a, h, p["cW3"], p["cb3"].reshape(1, -1), dinv, relu=False)

    tok = jnp.concatenate([gen, h], axis=0)
    tok = jnp.pad(tok, ((0, T_PAD - T), (0, 0)))
    wqkv = jnp.concatenate([p["Wq"], p["Wk"], p["Wv"]], axis=1)
    bqkv = jnp.concatenate([p["bq"], p["bk"], p["bv"]])[None, :]
    qkv = _qkv_proj(tok, wqkv, bqkv)
    xn = _attention(tok, qkv, p["Wo"], p["bo"].reshape(1, D),
                    p["fW1"], p["fb1"].reshape(1, -1),
                    p["fW2"], p["fb2"].reshape(1, -1),
                    p["ln_g"].reshape(1, D), p["ln_b"].reshape(1, D))

    cwp = jnp.pad(p["cls_W"], ((0, 0), (0, 124)))
    cbp = jnp.pad(p["cls_b"], (0, 124))[None, :]
    out = _head(xn, p["mm_W"], p["mm_b"].reshape(1, 128), cwp, cbp)
    return out[0:1, 0:4]
